# baseline (speedup 1.0000x reference)
import dataclasses
import functools

import jax
import jax.numpy as jnp
from jax import lax
from jax.experimental import pallas as pl
from jax.experimental.pallas import tpu as pltpu
from jax.experimental.pallas import tpu_sc as plsc

N = 1_000_000
HIDDEN = 256
EMB = 8
XDIM = 11
VOCAB = 1000
TROWS_PAD = 64

SC_CORES = 2
SC_SUBCORES = 16
SC_LANES = 16
SC_BLOCK = 1024
SC_CHUNK = SC_BLOCK * SC_CORES * SC_SUBCORES
N_PAD = ((N + SC_CHUNK - 1) // SC_CHUNK) * SC_CHUNK

BT = 8192

SC_LO = (0, 65536, 196608, 491520)
SC_LEN = (65536, 131072, 294912, 524288)
TC_OFF = (0, 8, 24, 60)
TC_GRID = (8, 16, 36, 63)


def _sc_gather(table_packed, idx_chunk, n_rows):
    mesh = plsc.VectorSubcoreMesh(core_axis_name="core",
                                  subcore_axis_name="subcore")
    cp = pltpu.CompilerParams()
    if "needs_layout_passes" in pltpu.CompilerParams.__dataclass_fields__:
        cp = dataclasses.replace(cp, needs_layout_passes=False)

    @functools.partial(
        pl.kernel,
        out_type=jax.ShapeDtypeStruct((EMB, n_rows), jnp.float32),
        mesh=mesh,
        scratch_types=[pltpu.VMEM((TROWS_PAD, 128), jnp.float32)],
        compiler_params=cp,
    )
    def gather_kernel(table_hbm, idx_hbm, out_hbm, table_v):
        pltpu.sync_copy(table_hbm, table_v)

        lanes = lax.iota(jnp.int32, SC_LANES)
        row_off = lanes >> 3
        cold = lanes & 7
        zero = jnp.zeros((SC_LANES,), jnp.int32)

        def body(idx_vmem, out_vmem):
            @pl.loop(0, SC_BLOCK, step=2)
            def _(i):
                ridx = plsc.load_gather(idx_vmem, [zero, i + row_off])
                tflat = (ridx << 3) + cold
                vals = plsc.load_gather(table_v, [tflat >> 7, tflat & 127])
                plsc.store_scatter(out_vmem, [cold, i + row_off], vals)

        pltpu.emit_pipeline(
            body,
            grid=(n_rows // SC_BLOCK,),
            in_specs=[pl.BlockSpec((1, SC_BLOCK), lambda i: (0, i))],
            out_specs=[pl.BlockSpec((EMB, SC_BLOCK), lambda i: (0, i))],
            core_axis_name=("core", "subcore"),
            dimension_semantics=(pltpu.PARALLEL,),
        )(idx_hbm, out_hbm)

    return gather_kernel(table_packed, idx_chunk)


def _dot_block(zebt_ref, xt_ref, wc_ref, b_ref, o_ref):
    h = jnp.concatenate(
        [zebt_ref[...].astype(jnp.bfloat16),
         xt_ref[...].astype(jnp.bfloat16)], axis=0)
    o_ref[...] = lax.dot_general(
        h, wc_ref[...],
        dimension_numbers=(((0,), (0,)), ((), ())),
        preferred_element_type=jnp.float32) + b_ref[...]


def _tc_body_first(zebt_ref, xt_ref, wc_ref, b_ref, o_ref):
    _dot_block(zebt_ref, xt_ref, wc_ref, b_ref, o_ref)


def _tc_body_chain(zebt_ref, xt_ref, wc_ref, b_ref, prev_ref, o_ref):
    del prev_ref
    _dot_block(zebt_ref, xt_ref, wc_ref, b_ref, o_ref)


def _tc_chunk(zebt_c, xt, wc, b2d, prev, off, grid_n):
    in_specs = [
        pl.BlockSpec((EMB, BT), lambda i: (0, i)),
        pl.BlockSpec((XDIM, BT), lambda i, o=off: (0, o + i)),
        pl.BlockSpec((EMB + XDIM, HIDDEN), lambda i: (0, 0)),
        pl.BlockSpec((1, HIDDEN), lambda i: (0, 0)),
    ]
    operands = [zebt_c, xt, wc, b2d]
    if prev is None:
        body, aliases = _tc_body_first, {}
    else:
        in_specs.append(pl.BlockSpec(memory_space=pltpu.MemorySpace.HBM))
        operands.append(prev)
        body, aliases = _tc_body_chain, {4: 0}
    return pl.pallas_call(
        body,
        grid=(grid_n,),
        in_specs=in_specs,
        out_specs=pl.BlockSpec((BT, HIDDEN), lambda i, o=off: (o + i, 0)),
        out_shape=jax.ShapeDtypeStruct((N, HIDDEN), jnp.float32),
        input_output_aliases=aliases,
    )(*operands)


def kernel(x, z, z_table, W, b):
    idx = jnp.pad(z.astype(jnp.int32), (0, N_PAD - N)).reshape(1, N_PAD)
    table_packed = jnp.pad(z_table.reshape(-1),
                           (0, TROWS_PAD * 128 - VOCAB * EMB)
                           ).reshape(TROWS_PAD, 128)
    wc = W.astype(jnp.bfloat16)
    b2d = b.reshape(1, HIDDEN)
    xt = x.T

    out = None
    for c in range(len(SC_LO)):
        lo, ln = SC_LO[c], SC_LEN[c]
        zebt_c = _sc_gather(table_packed, idx[:, lo:lo + ln], ln)
        out = _tc_chunk(zebt_c, xt, wc, b2d, out, TC_OFF[c], TC_GRID[c])
    return out


# speedup vs baseline: 1.0542x; 1.0542x over previous
import dataclasses
import functools

import jax
import jax.numpy as jnp
from jax import lax
from jax.experimental import pallas as pl
from jax.experimental.pallas import tpu as pltpu
from jax.experimental.pallas import tpu_sc as plsc

N = 1_000_000
HIDDEN = 256
EMB = 8
XDIM = 11
VOCAB = 1000
TROWS_PAD = 64

SC_CORES = 2
SC_SUBCORES = 16
SC_LANES = 16
SC_BLOCK = 1024
SC_CHUNK = SC_BLOCK * SC_CORES * SC_SUBCORES
N_PAD = ((N + SC_CHUNK - 1) // SC_CHUNK) * SC_CHUNK

BT = 8192

SC_LO = (0, 65536, 262144, 589824)
SC_LEN = (65536, 196608, 327680, 425984)
TC_OFF = (0, 8, 32, 72)
TC_GRID = (8, 24, 40, 51)


def _sc_gather(table_packed, idx_chunk, n_rows):
    mesh = plsc.VectorSubcoreMesh(core_axis_name="core",
                                  subcore_axis_name="subcore")
    cp = pltpu.CompilerParams()
    if "needs_layout_passes" in pltpu.CompilerParams.__dataclass_fields__:
        cp = dataclasses.replace(cp, needs_layout_passes=False)

    @functools.partial(
        pl.kernel,
        out_type=jax.ShapeDtypeStruct((EMB, n_rows), jnp.float32),
        mesh=mesh,
        scratch_types=[pltpu.VMEM((TROWS_PAD, 128), jnp.float32)],
        compiler_params=cp,
    )
    def gather_kernel(table_hbm, idx_hbm, out_hbm, table_v):
        pltpu.sync_copy(table_hbm, table_v)

        lanes = lax.iota(jnp.int32, SC_LANES)
        row_off = lanes >> 3
        cold = lanes & 7
        zero = jnp.zeros((SC_LANES,), jnp.int32)

        def body(idx_vmem, out_vmem):
            @pl.loop(0, SC_BLOCK, step=4)
            def _(i):
                pos0 = i + row_off
                pos1 = pos0 + 2
                ridx0 = plsc.load_gather(idx_vmem, [zero, pos0])
                ridx1 = plsc.load_gather(idx_vmem, [zero, pos1])
                tf0 = (ridx0 << 3) + cold
                tf1 = (ridx1 << 3) + cold
                v0 = plsc.load_gather(table_v, [tf0 >> 7, tf0 & 127])
                v1 = plsc.load_gather(table_v, [tf1 >> 7, tf1 & 127])
                plsc.store_scatter(out_vmem, [cold, pos0], v0)
                plsc.store_scatter(out_vmem, [cold, pos1], v1)

        pltpu.emit_pipeline(
            body,
            grid=(n_rows // SC_BLOCK,),
            in_specs=[pl.BlockSpec((1, SC_BLOCK), lambda i: (0, i))],
            out_specs=[pl.BlockSpec((EMB, SC_BLOCK), lambda i: (0, i))],
            core_axis_name=("core", "subcore"),
            dimension_semantics=(pltpu.PARALLEL,),
        )(idx_hbm, out_hbm)

    return gather_kernel(table_packed, idx_chunk)


def _dot_block(zebt_ref, xt_ref, wc_ref, b_ref, o_ref):
    h = jnp.concatenate(
        [zebt_ref[...].astype(jnp.bfloat16),
         xt_ref[...].astype(jnp.bfloat16)], axis=0)
    o_ref[...] = lax.dot_general(
        h, wc_ref[...],
        dimension_numbers=(((0,), (0,)), ((), ())),
        preferred_element_type=jnp.float32) + b_ref[...]


def _tc_body_first(zebt_ref, xt_ref, wc_ref, b_ref, o_ref):
    _dot_block(zebt_ref, xt_ref, wc_ref, b_ref, o_ref)


def _tc_body_chain(zebt_ref, xt_ref, wc_ref, b_ref, prev_ref, o_ref):
    del prev_ref
    _dot_block(zebt_ref, xt_ref, wc_ref, b_ref, o_ref)


def _tc_chunk(zebt_c, xt, wc, b2d, prev, off, grid_n):
    in_specs = [
        pl.BlockSpec((EMB, BT), lambda i: (0, i)),
        pl.BlockSpec((XDIM, BT), lambda i, o=off: (0, o + i)),
        pl.BlockSpec((EMB + XDIM, HIDDEN), lambda i: (0, 0)),
        pl.BlockSpec((1, HIDDEN), lambda i: (0, 0)),
    ]
    operands = [zebt_c, xt, wc, b2d]
    if prev is None:
        body, aliases = _tc_body_first, {}
    else:
        in_specs.append(pl.BlockSpec(memory_space=pltpu.MemorySpace.HBM))
        operands.append(prev)
        body, aliases = _tc_body_chain, {4: 0}
    return pl.pallas_call(
        body,
        grid=(grid_n,),
        in_specs=in_specs,
        out_specs=pl.BlockSpec((BT, HIDDEN), lambda i, o=off: (o + i, 0)),
        out_shape=jax.ShapeDtypeStruct((N, HIDDEN), jnp.float32),
        input_output_aliases=aliases,
    )(*operands)


def kernel(x, z, z_table, W, b):
    idx = jnp.pad(z.astype(jnp.int32), (0, N_PAD - N)).reshape(1, N_PAD)
    table_packed = jnp.pad(z_table.reshape(-1),
                           (0, TROWS_PAD * 128 - VOCAB * EMB)
                           ).reshape(TROWS_PAD, 128)
    wc = W.astype(jnp.bfloat16)
    b2d = b.reshape(1, HIDDEN)
    xt = x.T

    out = None
    for c in range(len(SC_LO)):
        lo, ln = SC_LO[c], SC_LEN[c]
        zebt_c = _sc_gather(table_packed, idx[:, lo:lo + ln], ln)
        out = _tc_chunk(zebt_c, xt, wc, b2d, out, TC_OFF[c], TC_GRID[c])
    return out
